# baseline (speedup 1.0000x reference)
import jax
import jax.numpy as jnp
from jax import lax
from jax.experimental import pallas as pl
from jax.experimental.pallas import tpu as pltpu
from jax.experimental.pallas import tpu_sc as plsc

V = 10000
E = 320000
FD = 512
NT = 32
R = 160
DP = 2
VP = DP * NT * R
EPT = E
CH = 2000
NGRP = CH // 16
NVR = FD // 16
_CP = pltpu.CompilerParams(needs_layout_passes=False)


def _make_step(has_p):
    def body(*refs):
        if has_p:
            (x2d, pfl, src_h, dst_h, ew_h, yfl,
             srcb, dstb, ewb, ksrc, kdst, kw, row0, row1, acc,
             sem0, sem1) = refs
        else:
            (x2d, src_h, dst_h, ew_h, yfl,
             srcb, dstb, ewb, ksrc, kdst, kw, row0, row1, acc,
             sem0, sem1) = refs
            pfl = None
        cc_ = lax.axis_index("c")
        ss_ = lax.axis_index("s")
        wid = ss_ * 2 + cc_
        iota = lax.iota(jnp.int32, 16)
        zero16 = jnp.zeros((16,), jnp.float32)

        def process(rowbuf, gp):
            def one_row(r, carry):
                e = gp + r
                wspl = plsc.load_gather(kw, [jnp.full((16,), e, jnp.int32)])
                dspl = plsc.load_gather(kdst, [jnp.full((16,), e, jnp.int32)])
                fb = dspl * FD + iota
                for f in range(NVR):
                    plsc.addupdate_scatter(
                        acc, [fb + (f * 16)],
                        rowbuf[r, pl.ds(f * 16, 16)] * wspl)
                return carry
            lax.fori_loop(0, 16, one_row, 0)

        def issue(g, buf, sem):
            idx = ksrc[pl.ds(g * 16, 16)]
            pltpu.async_copy(x2d.at[idx], buf, sem)

        def wait(buf, sem):
            pltpu.make_async_copy(x2d.at[pl.ds(0, 16)], buf, sem).wait()

        def dst_pass(p, carry0):
            base = (p * NT + wid) * R
            fbase = base * FD

            if has_p:
                pltpu.sync_copy(pfl.at[pl.ds(fbase, R * FD)], acc)

            def init_blk(k, carry):
                for u in range(16):
                    sl = pl.ds(k * 256 + u * 16, 16)
                    if has_p:
                        acc[sl] = -acc[sl]
                    else:
                        acc[sl] = zero16
                return carry
            lax.fori_loop(0, R * FD // 256, init_blk, 0)

            def chunk_body(cc, carry):
                eo = pl.multiple_of(cc * CH, 8)
                pltpu.sync_copy(src_h.at[pl.ds(eo, CH)], srcb)
                pltpu.sync_copy(dst_h.at[pl.ds(eo, CH)], dstb)
                pltpu.sync_copy(ew_h.at[pl.ds(eo, CH)], ewb)

                def filt(g, rpos):
                    off = g * 16
                    d16 = dstb[pl.ds(off, 16)]
                    s16 = srcb[pl.ds(off, 16)]
                    w16 = ewb[pl.ds(off, 16)]
                    dloc = d16 - base
                    m = (dloc >= 0) & (dloc < R)
                    mi = m.astype(jnp.int32)
                    tgt = rpos + plsc.cumsum(mi) - 1
                    plsc.store_scatter(ksrc, [tgt], s16, mask=m)
                    plsc.store_scatter(kdst, [tgt], dloc, mask=m)
                    plsc.store_scatter(kw, [tgt], w16, mask=m)
                    return rpos + jnp.sum(mi)

                rpos = lax.fori_loop(0, NGRP, filt, jnp.int32(0))

                padn = lax.rem(32 - lax.rem(rpos, 32), 32)
                for pi in range(2):
                    mpad = iota < (padn - pi * 16)
                    tp = rpos + pi * 16 + iota
                    plsc.store_scatter(ksrc, [tp],
                                       jnp.zeros((16,), jnp.int32), mask=mpad)
                    plsc.store_scatter(kdst, [tp],
                                       jnp.zeros((16,), jnp.int32), mask=mpad)
                    plsc.store_scatter(kw, [tp], zero16, mask=mpad)
                ngr = (rpos + padn) // 16

                @pl.when(ngr > 0)
                def _():
                    issue(0, row0, sem0)

                    def proc2(j2, carry2):
                        g0 = j2 * 2
                        issue(g0 + 1, row1, sem1)
                        wait(row0, sem0)
                        process(row0, g0 * 16)

                        @pl.when(g0 + 2 < ngr)
                        def _():
                            issue(g0 + 2, row0, sem0)
                        wait(row1, sem1)
                        process(row1, (g0 + 1) * 16)
                        return carry2

                    lax.fori_loop(0, ngr // 2, proc2, 0)
                return carry

            lax.fori_loop(0, EPT // CH, chunk_body, 0)

            pltpu.sync_copy(acc, yfl.at[pl.ds(fbase, R * FD)])
            return carry0

        lax.fori_loop(0, DP, dst_pass, 0)

    return pl.kernel(
        body,
        out_type=jax.ShapeDtypeStruct((VP * FD,), jnp.float32),
        mesh=plsc.VectorSubcoreMesh(core_axis_name="c", subcore_axis_name="s"),
        compiler_params=_CP,
        scratch_types=[
            pltpu.VMEM((CH,), jnp.int32),
            pltpu.VMEM((CH,), jnp.int32),
            pltpu.VMEM((CH,), jnp.float32),
            pltpu.VMEM((CH + 32,), jnp.int32),
            pltpu.VMEM((CH + 32,), jnp.int32),
            pltpu.VMEM((CH + 32,), jnp.float32),
            pltpu.VMEM((16, FD), jnp.float32),
            pltpu.VMEM((16, FD), jnp.float32),
            pltpu.VMEM((R * FD,), jnp.float32),
            pltpu.SemaphoreType.DMA,
            pltpu.SemaphoreType.DMA,
        ],
    )


_step_first = _make_step(False)
_step_next = _make_step(True)


def _contract_body(z_ref, w_ref, b_ref, o_ref):
    acc = jnp.zeros(o_ref.shape, jnp.float32)
    for k in range(z_ref.shape[0]):
        acc = acc + jnp.dot(z_ref[k], w_ref[k], preferred_element_type=jnp.float32)
    o_ref[...] = acc + b_ref[...]


def kernel(x, edge_index, edge_weight, weight, bias):
    B, CIN, Vn = x.shape
    K, _, COUT = weight.shape
    dst = edge_index[0]
    src = edge_index[1]

    x0 = jnp.transpose(x, (2, 0, 1)).reshape(Vn, B * CIN)
    x0 = jnp.pad(x0, ((0, VP - Vn), (0, 0)))
    x0f = x0.reshape(-1)
    ew2 = edge_weight * 2.0

    y1f = _step_first(x0, src, dst, edge_weight)
    flats = [x0f, y1f]
    prevf, curf = x0f, y1f
    for _ in range(2, K):
        cur2d = curf.reshape(VP, FD)
        nf = _step_next(cur2d, prevf, src, dst, ew2)
        flats.append(nf)
        prevf, curf = curf, nf

    z = jnp.stack(flats).reshape(K, VP, FD)[:, :Vn].reshape(K, B * Vn, CIN)

    TM = 2000
    rows = B * Vn
    out = pl.pallas_call(
        _contract_body,
        grid=(rows // TM,),
        in_specs=[
            pl.BlockSpec((K, TM, CIN), lambda i: (0, i, 0)),
            pl.BlockSpec((K, CIN, COUT), lambda i: (0, 0, 0)),
            pl.BlockSpec((1, COUT), lambda i: (0, 0)),
        ],
        out_specs=pl.BlockSpec((TM, COUT), lambda i: (i, 0)),
        out_shape=jax.ShapeDtypeStruct((rows, COUT), jnp.float32),
    )(z, weight, bias.reshape(1, COUT))

    out = out.reshape(Vn, B, COUT)
    return jnp.transpose(out, (1, 2, 0))


# speedup vs baseline: 3.3170x; 3.3170x over previous
import jax
import jax.numpy as jnp
from jax import lax
from jax.experimental import pallas as pl
from jax.experimental.pallas import tpu as pltpu
from jax.experimental.pallas import tpu_sc as plsc

V = 10000
E = 320000
FD = 512
NT = 32
R = 160
DP = 2
NRID = DP * NT
VP = NRID * R
RIDMUL = 13108
RIDSHIFT = 21
CH_B = 4000
NGRP_B = CH_B // 16
LCAP = CH_B + 48
RCAP = 328000
CH_P = 8000
NVR = FD // 16
_CP = pltpu.CompilerParams(needs_layout_passes=False)


def _build_body(src_h, dst_h, ew_h, lsrc, ldst, lw, cnts,
                srcb, dstb, ewb, bs0, bd0, bw0, bs1, bd1, bw1, cvb):
    cc_ = lax.axis_index("c")
    ss_ = lax.axis_index("s")
    wid = ss_ * 2 + cc_
    iota = lax.iota(jnp.int32, 16)
    bufs = ((bs0, bd0, bw0), (bs1, bd1, bw1))

    def chunk_body(cc, carry):
        l0, l1, w0, w1, t0, t1 = carry
        eo = pl.multiple_of(cc * CH_B, 8)
        pltpu.sync_copy(src_h.at[pl.ds(eo, CH_B)], srcb)
        pltpu.sync_copy(dst_h.at[pl.ds(eo, CH_B)], dstb)
        pltpu.sync_copy(ew_h.at[pl.ds(eo, CH_B)], ewb)

        def filt(g, lc):
            l0c, l1c, t0c, t1c = lc
            off = g * 16
            d16 = dstb[pl.ds(off, 16)]
            s16 = srcb[pl.ds(off, 16)]
            w16 = ewb[pl.ds(off, 16)]
            rid16 = lax.shift_right_logical(d16 * RIDMUL, RIDSHIFT)
            mlow = (rid16 & 31) == wid
            m0 = mlow & (rid16 < NT)
            m1 = mlow & (rid16 >= NT)
            c0 = jnp.sum(m0.astype(jnp.int32))
            c1 = jnp.sum(m1.astype(jnp.int32))

            tgt0 = l0c + plsc.cumsum(m0.astype(jnp.int32)) - 1
            plsc.store_scatter(bs0, [tgt0], s16, mask=m0)
            plsc.store_scatter(bd0, [tgt0], d16, mask=m0)
            plsc.store_scatter(bw0, [tgt0], w16, mask=m0)
            tgt1 = l1c + plsc.cumsum(m1.astype(jnp.int32)) - 1
            plsc.store_scatter(bs1, [tgt1], s16, mask=m1)
            plsc.store_scatter(bd1, [tgt1], d16, mask=m1)
            plsc.store_scatter(bw1, [tgt1], w16, mask=m1)
            return (l0c + c0, l1c + c1, t0c + c0, t1c + c1)

        l0, l1, t0, t1 = lax.fori_loop(0, NGRP_B, filt, (l0, l1, t0, t1))

        new = []
        for p, (lp, wp) in enumerate(((l0, w0), (l1, w1))):
            sb, db, wb = bufs[p]
            rbase = (p * NT + wid) * RCAP
            woff = pl.multiple_of(rbase + wp, 8)
            pltpu.sync_copy(sb.at[pl.ds(0, LCAP)], lsrc.at[pl.ds(woff, LCAP)])
            pltpu.sync_copy(db.at[pl.ds(0, LCAP)], ldst.at[pl.ds(woff, LCAP)])
            pltpu.sync_copy(wb.at[pl.ds(0, LCAP)], lw.at[pl.ds(woff, LCAP)])
            f16 = (lp // 16) * 16
            rem = lp - f16
            mrem = iota < rem
            for b in (sb, db):
                tv = b[pl.ds(f16, 16)]
                plsc.store_scatter(b, [iota], tv, mask=mrem)
            tvw = wb[pl.ds(f16, 16)]
            plsc.store_scatter(wb, [iota], tvw, mask=mrem)
            new.append((rem, wp + f16))
        return (new[0][0], new[1][0], new[0][1], new[1][1], t0, t1)

    l0, l1, w0, w1, t0, t1 = lax.fori_loop(0, E // CH_B, chunk_body,
                                   (jnp.int32(0), jnp.int32(0),
                                    jnp.int32(0), jnp.int32(0),
                                    jnp.int32(0), jnp.int32(0)))

    for p, (lp, wp, tp_) in enumerate(((l0, w0, t0), (l1, w1, t1))):
        sb, db, wb = bufs[p]
        rid = p * NT + wid
        rbase = rid * RCAP
        dnull = jnp.full((16,), 0, jnp.int32) + rid * R
        for pi in range(2):
            tp = lp + pi * 16 + iota
            plsc.store_scatter(sb, [tp], jnp.zeros((16,), jnp.int32))
            plsc.store_scatter(db, [tp], dnull)
            plsc.store_scatter(wb, [tp], jnp.zeros((16,), jnp.float32))
        woff = pl.multiple_of(rbase + wp, 8)
        pltpu.sync_copy(sb.at[pl.ds(0, LCAP)], lsrc.at[pl.ds(woff, LCAP)])
        pltpu.sync_copy(db.at[pl.ds(0, LCAP)], ldst.at[pl.ds(woff, LCAP)])
        pltpu.sync_copy(wb.at[pl.ds(0, LCAP)], lw.at[pl.ds(woff, LCAP)])
        cnt = wp + lp
        cvb[pl.ds(0, 16)] = jnp.full((16,), 0, jnp.int32) + cnt
        pltpu.sync_copy(cvb, cnts.at[rid])


_build = pl.kernel(
    _build_body,
    out_type=(jax.ShapeDtypeStruct((NRID * RCAP,), jnp.int32),
              jax.ShapeDtypeStruct((NRID * RCAP,), jnp.int32),
              jax.ShapeDtypeStruct((NRID * RCAP,), jnp.float32),
              jax.ShapeDtypeStruct((NRID, 16), jnp.int32)),
    mesh=plsc.VectorSubcoreMesh(core_axis_name="c", subcore_axis_name="s"),
    compiler_params=_CP,
    scratch_types=[
        pltpu.VMEM((CH_B,), jnp.int32),
        pltpu.VMEM((CH_B,), jnp.int32),
        pltpu.VMEM((CH_B,), jnp.float32),
        pltpu.VMEM((LCAP,), jnp.int32),
        pltpu.VMEM((LCAP,), jnp.int32),
        pltpu.VMEM((LCAP,), jnp.float32),
        pltpu.VMEM((LCAP,), jnp.int32),
        pltpu.VMEM((LCAP,), jnp.int32),
        pltpu.VMEM((LCAP,), jnp.float32),
        pltpu.VMEM((16,), jnp.int32),
    ],
)


def _make_step(has_p):
    def body(*refs):
        if has_p:
            (x2d, pfl, lsrc, ldst, lw, cnts, yfl,
             lsrcb, ldstb, lwb, row0, row1, acc, cvb, sem0, sem1) = refs
        else:
            (x2d, lsrc, ldst, lw, cnts, yfl,
             lsrcb, ldstb, lwb, row0, row1, acc, cvb, sem0, sem1) = refs
            pfl = None
        cc_ = lax.axis_index("c")
        ss_ = lax.axis_index("s")
        wid = ss_ * 2 + cc_
        iota = lax.iota(jnp.int32, 16)
        zero16 = jnp.zeros((16,), jnp.float32)

        def issue(g, buf, sem):
            idx = lsrcb[pl.ds(g * 16, 16)]
            pltpu.async_copy(x2d.at[idx], buf, sem)

        def wait(buf, sem):
            pltpu.make_async_copy(x2d.at[pl.ds(0, 16)], buf, sem).wait()

        def dst_pass(p, carry0):
            rid = p * NT + wid
            base = rid * R
            fbase = base * FD
            rbase = rid * RCAP

            if has_p:
                pltpu.sync_copy(pfl.at[pl.ds(fbase, R * FD)], acc)

            def init_blk(k, carry):
                for u in range(16):
                    sl = pl.ds(k * 256 + u * 16, 16)
                    if has_p:
                        acc[sl] = -acc[sl]
                    else:
                        acc[sl] = zero16
                return carry
            lax.fori_loop(0, R * FD // 256, init_blk, 0)

            def process(rowbuf, gp):
                def one_row(r, carry):
                    e = gp + r
                    esp = jnp.full((16,), e, jnp.int32)
                    wspl = plsc.load_gather(lwb, [esp])
                    if has_p:
                        wspl = wspl * 2.0
                    dspl = plsc.load_gather(ldstb, [esp])
                    fb = (dspl - base) * FD + iota
                    for f in range(NVR):
                        plsc.addupdate_scatter(
                            acc, [fb + (f * 16)],
                            rowbuf[r, pl.ds(f * 16, 16)] * wspl)
                    return carry
                lax.fori_loop(0, 16, one_row, 0)

            pltpu.sync_copy(cnts.at[rid], cvb)
            cnt = jnp.sum(jnp.where(iota == 0, cvb[pl.ds(0, 16)], 0))
            nch = (cnt + CH_P - 1) // CH_P

            def chunk_body(ch, carry):
                off = pl.multiple_of(rbase + ch * CH_P, 8)
                pltpu.sync_copy(lsrc.at[pl.ds(off, CH_P)], lsrcb)
                pltpu.sync_copy(ldst.at[pl.ds(off, CH_P)], ldstb)
                pltpu.sync_copy(lw.at[pl.ds(off, CH_P)], lwb)
                ng = ((jnp.minimum(cnt - ch * CH_P, CH_P) + 31) // 32) * 2

                @pl.when(ng > 0)
                def _():
                    issue(0, row0, sem0)

                    def proc2(j2, carry2):
                        g0 = j2 * 2
                        issue(g0 + 1, row1, sem1)
                        wait(row0, sem0)
                        process(row0, g0 * 16)

                        @pl.when(g0 + 2 < ng)
                        def _():
                            issue(g0 + 2, row0, sem0)
                        wait(row1, sem1)
                        process(row1, (g0 + 1) * 16)
                        return carry2

                    lax.fori_loop(0, ng // 2, proc2, 0)
                return carry

            lax.fori_loop(0, nch, chunk_body, 0)

            pltpu.sync_copy(acc, yfl.at[pl.ds(fbase, R * FD)])
            return carry0

        lax.fori_loop(0, DP, dst_pass, 0)

    return pl.kernel(
        body,
        out_type=jax.ShapeDtypeStruct((VP * FD,), jnp.float32),
        mesh=plsc.VectorSubcoreMesh(core_axis_name="c", subcore_axis_name="s"),
        compiler_params=_CP,
        scratch_types=[
            pltpu.VMEM((CH_P,), jnp.int32),
            pltpu.VMEM((CH_P,), jnp.int32),
            pltpu.VMEM((CH_P,), jnp.float32),
            pltpu.VMEM((16, FD), jnp.float32),
            pltpu.VMEM((16, FD), jnp.float32),
            pltpu.VMEM((R * FD,), jnp.float32),
            pltpu.VMEM((16,), jnp.int32),
            pltpu.SemaphoreType.DMA,
            pltpu.SemaphoreType.DMA,
        ],
    )


_step_first = _make_step(False)
_step_next = _make_step(True)


def _contract_body(z_ref, w_ref, b_ref, o_ref):
    acc = jnp.zeros(o_ref.shape, jnp.float32)
    for k in range(z_ref.shape[0]):
        acc = acc + jnp.dot(z_ref[k], w_ref[k], preferred_element_type=jnp.float32)
    o_ref[...] = acc + b_ref[...]


def kernel(x, edge_index, edge_weight, weight, bias):
    B, CIN, Vn = x.shape
    K, _, COUT = weight.shape
    dst = edge_index[0]
    src = edge_index[1]

    x0 = jnp.transpose(x, (2, 0, 1)).reshape(Vn, B * CIN)
    x0 = jnp.pad(x0, ((0, VP - Vn), (0, 0)))
    x0f = x0.reshape(-1)
    lsrc, ldst, lw, cnts = _build(src, dst, edge_weight)

    y1f = _step_first(x0, lsrc, ldst, lw, cnts)
    flats = [x0f, y1f]
    prevf, curf = x0f, y1f
    for _ in range(2, K):
        cur2d = curf.reshape(VP, FD)
        nf = _step_next(cur2d, prevf, lsrc, ldst, lw, cnts)
        flats.append(nf)
        prevf, curf = curf, nf

    z = jnp.stack(flats).reshape(K, VP, FD)[:, :Vn].reshape(K, B * Vn, CIN)

    TM = 2000
    rows = B * Vn
    out = pl.pallas_call(
        _contract_body,
        grid=(rows // TM,),
        in_specs=[
            pl.BlockSpec((K, TM, CIN), lambda i: (0, i, 0)),
            pl.BlockSpec((K, CIN, COUT), lambda i: (0, 0, 0)),
            pl.BlockSpec((1, COUT), lambda i: (0, 0)),
        ],
        out_specs=pl.BlockSpec((TM, COUT), lambda i: (i, 0)),
        out_shape=jax.ShapeDtypeStruct((rows, COUT), jnp.float32),
    )(z, weight, bias.reshape(1, COUT))

    out = out.reshape(Vn, B, COUT)
    return jnp.transpose(out, (1, 2, 0))
